# baseline (speedup 1.0000x reference)
import functools

import jax
import jax.numpy as jnp
from jax import lax
from jax.experimental import pallas as pl
from jax.experimental.pallas import tpu as pltpu
from jax.experimental.pallas import tpu_sc as plsc

_N, _E, _D = 10000, 320000, 128
_NC, _NS = 2, 16
_NTILES = _NC * _NS
_EPT = _E // _NTILES
_K = 128
_NCHUNK = 79
_PAD = _NCHUNK * _K - _EPT
_PH1 = 40
_PH2 = _NCHUNK - _PH1
_ZROWS = _N // _NS


def _make_aggregate():
  mesh = plsc.VectorSubcoreMesh(core_axis_name="c", subcore_axis_name="s")

  @functools.partial(
      pl.kernel,
      mesh=mesh,
      out_type=jax.ShapeDtypeStruct((_NC, _NS, _ZROWS, _D), jnp.float32),
      scratch_types=[
          pltpu.VMEM((2 * _PH1, _K), jnp.int32),
          pltpu.VMEM((_K, _D), jnp.float32),
          pltpu.VMEM((_K, _D), jnp.float32),
          pltpu.VMEM_SHARED((_N + 8, _D), jnp.float32),
          pltpu.SemaphoreType.DMA,
          pltpu.SemaphoreType.DMA,
          pltpu.SemaphoreType.DMA,
      ],
  )
  def agg_kernel(h_hbm, eidx_hbm, zero_hbm, out_hbm,
                 idx_v, rows0_v, rows1_v, agg_sh, sem0, sem1, semz):
    c = lax.axis_index("c")
    s = lax.axis_index("s")
    wid = c * _NS + s

    def gather(rr, buf, sem):
      pltpu.async_copy(h_hbm.at[idx_v.at[rr]], buf, sem)

    def drain(buf, sem):
      pltpu.make_async_copy(h_hbm.at[idx_v.at[0]], buf, sem).wait()

    def scatter_add(rr, buf):
      pltpu.sync_copy(buf, agg_sh.at[idx_v.at[rr]], add=True)

    def pair_phase(npairs):
      def pair(i, carry):
        drain(rows0_v, sem0)
        scatter_add(4 * i + 1, rows0_v)
        gather(4 * i + 4, rows0_v, sem0)
        drain(rows1_v, sem1)
        scatter_add(4 * i + 3, rows1_v)
        gather(4 * i + 6, rows1_v, sem1)
        return carry

      lax.fori_loop(0, npairs, pair, 0)

    idx_stage = pltpu.async_copy(eidx_hbm.at[wid, pl.ds(0, 2 * _PH1)], idx_v,
                                 sem0)
    zero_stage = pltpu.async_copy(zero_hbm,
                                  agg_sh.at[pl.ds(s * _ZROWS, _ZROWS)], semz)
    idx_stage.wait()
    gather(0, rows0_v, sem0)
    gather(2, rows1_v, sem1)
    zero_stage.wait()
    plsc.subcore_barrier()

    pair_phase(_PH1 // 2 - 1)
    drain(rows0_v, sem0)
    scatter_add(2 * (_PH1 - 2) + 1, rows0_v)
    drain(rows1_v, sem1)
    scatter_add(2 * (_PH1 - 1) + 1, rows1_v)

    pltpu.sync_copy(eidx_hbm.at[wid, pl.ds(2 * _PH1, 2 * _PH2)],
                    idx_v.at[pl.ds(0, 2 * _PH2)])
    gather(0, rows0_v, sem0)
    gather(2, rows1_v, sem1)

    pair_phase(_PH2 // 2 - 1)
    drain(rows0_v, sem0)
    scatter_add(2 * (_PH2 - 3) + 1, rows0_v)
    gather(2 * (_PH2 - 1), rows0_v, sem0)
    drain(rows1_v, sem1)
    scatter_add(2 * (_PH2 - 2) + 1, rows1_v)
    drain(rows0_v, sem0)
    scatter_add(2 * (_PH2 - 1) + 1, rows0_v)

    plsc.subcore_barrier()
    pltpu.sync_copy(agg_sh.at[pl.ds(s * _ZROWS, _ZROWS)], out_hbm.at[c, s])

  return agg_kernel


_AGGREGATE = _make_aggregate()

_BLK = 400


def _mlp_body(p_ref, h_ref, wa_ref, ba_ref, wb_ref, bb_ref, o_ref):
  t = p_ref[0] + p_ref[1] + h_ref[...]
  t = jnp.dot(t, wa_ref[...], preferred_element_type=jnp.float32,
              precision=lax.Precision.HIGHEST) + ba_ref[...]
  t = jnp.maximum(t, 0.0)
  t = jnp.dot(t, wb_ref[...], preferred_element_type=jnp.float32,
              precision=lax.Precision.HIGHEST) + bb_ref[...]
  o_ref[...] = jnp.maximum(t, 0.0)


def _mlp(parts, h, wa_t, ba, wb_t, bb):
  return pl.pallas_call(
      _mlp_body,
      grid=(_N // _BLK,),
      in_specs=[
          pl.BlockSpec((_NC, _BLK, _D), lambda i: (0, i, 0)),
          pl.BlockSpec((_BLK, _D), lambda i: (i, 0)),
          pl.BlockSpec((_D, _D), lambda i: (0, 0)),
          pl.BlockSpec((1, _D), lambda i: (0, 0)),
          pl.BlockSpec((_D, _D), lambda i: (0, 0)),
          pl.BlockSpec((1, _D), lambda i: (0, 0)),
      ],
      out_specs=pl.BlockSpec((_BLK, _D), lambda i: (i, 0)),
      out_shape=jax.ShapeDtypeStruct((_N, _D), jnp.float32),
  )(parts, h, wa_t, ba, wb_t, bb)


def kernel(x, edge_index, W0, b0, W1, b1, W2, b2, W3, b3, W4, b4, W5, b5):
  src = jnp.pad(edge_index[0].reshape(_NTILES, _EPT), ((0, 0), (0, _PAD)))
  dst = jnp.pad(edge_index[1].reshape(_NTILES, _EPT), ((0, 0), (0, _PAD)),
                constant_values=_N)
  eidx = jnp.stack([src.reshape(_NTILES, _NCHUNK, _K),
                    dst.reshape(_NTILES, _NCHUNK, _K)],
                   axis=2).reshape(_NTILES, 2 * _NCHUNK, _K)
  zeros_blk = jnp.zeros((_ZROWS, _D), jnp.float32)
  h = x
  for wa, ba, wb, bb in ((W0, b0, W1, b1), (W2, b2, W3, b3), (W4, b4, W5, b5)):
    parts = _AGGREGATE(h, eidx, zeros_blk).reshape(_NC, _N, _D)
    h = _mlp(parts, h, wa.T, ba.reshape(1, _D), wb.T, bb.reshape(1, _D))
  return h


# speedup vs baseline: 1.8351x; 1.8351x over previous
import functools

import jax
import jax.numpy as jnp
from jax import lax
from jax.experimental import pallas as pl
from jax.experimental.pallas import tpu as pltpu
from jax.experimental.pallas import tpu_sc as plsc

_N, _E, _D = 10000, 320000, 128
_NC, _NS = 2, 16
_NTILES = _NC * _NS
_EPT = _E // _NTILES
_K = 125
_NCHUNK = _EPT // _K
_PH = _NCHUNK // 2
_ZROWS = _N // _NS


def _make_aggregate():
  mesh = plsc.VectorSubcoreMesh(core_axis_name="c", subcore_axis_name="s")

  @functools.partial(
      pl.kernel,
      mesh=mesh,
      out_type=jax.ShapeDtypeStruct((_NC, _NS, _ZROWS, _D), jnp.float32),
      scratch_types=[
          pltpu.VMEM((2 * _PH, _K), jnp.int32),
          pltpu.VMEM((_K, _D), jnp.float32),
          pltpu.VMEM((_K, _D), jnp.float32),
          pltpu.VMEM_SHARED((_N, _D), jnp.float32),
          pltpu.SemaphoreType.DMA,
          pltpu.SemaphoreType.DMA,
          pltpu.SemaphoreType.DMA,
      ],
  )
  def agg_kernel(h_hbm, eidx_hbm, zero_hbm, out_hbm,
                 idx_v, rows0_v, rows1_v, agg_sh, sem0, sem1, semz):
    c = lax.axis_index("c")
    s = lax.axis_index("s")
    wid = c * _NS + s

    def gather(rr, buf, sem):
      pltpu.async_copy(h_hbm.at[idx_v.at[rr]], buf, sem)

    def drain(buf, sem):
      pltpu.make_async_copy(h_hbm.at[idx_v.at[0]], buf, sem).wait()

    def scatter_add(rr, buf):
      pltpu.sync_copy(buf, agg_sh.at[idx_v.at[rr]], add=True)

    def pair_phase(npairs):
      def pair(i, carry):
        drain(rows0_v, sem0)
        scatter_add(4 * i + 1, rows0_v)
        gather(4 * i + 4, rows0_v, sem0)
        drain(rows1_v, sem1)
        scatter_add(4 * i + 3, rows1_v)
        gather(4 * i + 6, rows1_v, sem1)
        return carry

      lax.fori_loop(0, npairs, pair, 0)

    idx_stage = pltpu.async_copy(eidx_hbm.at[wid, pl.ds(0, 2 * _PH)], idx_v,
                                 sem0)
    zero_stage = pltpu.async_copy(zero_hbm,
                                  agg_sh.at[pl.ds(s * _ZROWS, _ZROWS)], semz)
    idx_stage.wait()
    gather(0, rows0_v, sem0)
    gather(2, rows1_v, sem1)
    zero_stage.wait()
    plsc.subcore_barrier()

    pair_phase(_PH // 2 - 1)
    drain(rows0_v, sem0)
    scatter_add(2 * (_PH - 2) + 1, rows0_v)
    drain(rows1_v, sem1)
    scatter_add(2 * (_PH - 1) + 1, rows1_v)

    pltpu.sync_copy(eidx_hbm.at[wid, pl.ds(2 * _PH, 2 * _PH)], idx_v)
    gather(0, rows0_v, sem0)
    gather(2, rows1_v, sem1)

    pair_phase(_PH // 2 - 1)
    drain(rows0_v, sem0)
    scatter_add(2 * (_PH - 2) + 1, rows0_v)
    drain(rows1_v, sem1)
    scatter_add(2 * (_PH - 1) + 1, rows1_v)

    plsc.subcore_barrier()
    pltpu.sync_copy(agg_sh.at[pl.ds(s * _ZROWS, _ZROWS)], out_hbm.at[c, s])

  return agg_kernel


_AGGREGATE = _make_aggregate()

_BLK = 400


def _mlp_body(p_ref, h_ref, wa_ref, ba_ref, wb_ref, bb_ref, o_ref):
  t = p_ref[0] + p_ref[1] + h_ref[...]
  t = jnp.dot(t, wa_ref[...], preferred_element_type=jnp.float32,
              precision=lax.Precision.HIGHEST) + ba_ref[...]
  t = jnp.maximum(t, 0.0)
  t = jnp.dot(t, wb_ref[...], preferred_element_type=jnp.float32,
              precision=lax.Precision.HIGHEST) + bb_ref[...]
  o_ref[...] = jnp.maximum(t, 0.0)


def _mlp(parts, h, wa_t, ba, wb_t, bb):
  return pl.pallas_call(
      _mlp_body,
      grid=(_N // _BLK,),
      in_specs=[
          pl.BlockSpec((_NC, _BLK, _D), lambda i: (0, i, 0)),
          pl.BlockSpec((_BLK, _D), lambda i: (i, 0)),
          pl.BlockSpec((_D, _D), lambda i: (0, 0)),
          pl.BlockSpec((1, _D), lambda i: (0, 0)),
          pl.BlockSpec((_D, _D), lambda i: (0, 0)),
          pl.BlockSpec((1, _D), lambda i: (0, 0)),
      ],
      out_specs=pl.BlockSpec((_BLK, _D), lambda i: (i, 0)),
      out_shape=jax.ShapeDtypeStruct((_N, _D), jnp.float32),
  )(parts, h, wa_t, ba, wb_t, bb)


def kernel(x, edge_index, W0, b0, W1, b1, W2, b2, W3, b3, W4, b4, W5, b5):
  eidx = jnp.stack([edge_index[0].reshape(_NTILES, _NCHUNK, _K),
                    edge_index[1].reshape(_NTILES, _NCHUNK, _K)],
                   axis=2).reshape(_NTILES, 2 * _NCHUNK, _K)
  zeros_blk = jnp.zeros((_ZROWS, _D), jnp.float32)
  h = x
  for wa, ba, wb, bb in ((W0, b0, W1, b1), (W2, b2, W3, b3), (W4, b4, W5, b5)):
    parts = _AGGREGATE(h, eidx, zeros_blk).reshape(_NC, _N, _D)
    h = _mlp(parts, h, wa.T, ba.reshape(1, _D), wb.T, bb.reshape(1, _D))
  return h


# speedup vs baseline: 2.2079x; 1.2031x over previous
import functools

import jax
import jax.numpy as jnp
from jax import lax
from jax.experimental import pallas as pl
from jax.experimental.pallas import tpu as pltpu
from jax.experimental.pallas import tpu_sc as plsc

_N, _E, _D = 10000, 320000, 128
_NC, _NS = 2, 16
_NTILES = _NC * _NS
_EPT = _E // _NTILES
_K = 125
_NCHUNK = _EPT // _K
_PH = _NCHUNK // 2
_ZROWS = _N // _NS


def _make_aggregate():
  mesh = plsc.VectorSubcoreMesh(core_axis_name="c", subcore_axis_name="s")

  @functools.partial(
      pl.kernel,
      mesh=mesh,
      out_type=jax.ShapeDtypeStruct((_NC, _N, _D), jnp.float32),
      scratch_types=[
          pltpu.VMEM((2 * _PH, _K), jnp.int32),
          pltpu.VMEM((_K, _D), jnp.float32),
          pltpu.VMEM((_K, _D), jnp.float32),
          pltpu.VMEM_SHARED((_N, _D), jnp.float32),
          pltpu.SemaphoreType.DMA,
          pltpu.SemaphoreType.DMA,
          pltpu.SemaphoreType.DMA,
      ],
  )
  def agg_kernel(h_hbm, eidx_hbm, zero_hbm, out_hbm,
                 idx_v, rows0_v, rows1_v, agg_sh, sem0, sem1, semz):
    c = lax.axis_index("c")
    s = lax.axis_index("s")
    wid = c * _NS + s

    def gather(rr, buf, sem):
      pltpu.async_copy(h_hbm.at[idx_v.at[rr]], buf, sem)

    def drain(buf, sem):
      pltpu.make_async_copy(h_hbm.at[idx_v.at[0]], buf, sem).wait()

    def scatter_add(rr, buf):
      pltpu.sync_copy(buf, agg_sh.at[idx_v.at[rr]], add=True)

    def pair_phase(npairs):
      def pair(i, carry):
        drain(rows0_v, sem0)
        scatter_add(4 * i + 1, rows0_v)
        gather(4 * i + 4, rows0_v, sem0)
        drain(rows1_v, sem1)
        scatter_add(4 * i + 3, rows1_v)
        gather(4 * i + 6, rows1_v, sem1)
        return carry

      lax.fori_loop(0, npairs, pair, 0)

    idx_stage = pltpu.async_copy(eidx_hbm.at[wid, pl.ds(0, 2 * _PH)], idx_v,
                                 sem0)
    zero_stage = pltpu.async_copy(zero_hbm,
                                  agg_sh.at[pl.ds(s * _ZROWS, _ZROWS)], semz)
    idx_stage.wait()
    gather(0, rows0_v, sem0)
    gather(2, rows1_v, sem1)
    zero_stage.wait()
    plsc.subcore_barrier()

    pair_phase(_PH // 2 - 1)
    drain(rows0_v, sem0)
    scatter_add(2 * (_PH - 2) + 1, rows0_v)
    drain(rows1_v, sem1)
    scatter_add(2 * (_PH - 1) + 1, rows1_v)

    pltpu.sync_copy(eidx_hbm.at[wid, pl.ds(2 * _PH, 2 * _PH)], idx_v)
    gather(0, rows0_v, sem0)
    gather(2, rows1_v, sem1)

    pair_phase(_PH // 2 - 1)
    drain(rows0_v, sem0)
    scatter_add(2 * (_PH - 2) + 1, rows0_v)
    drain(rows1_v, sem1)
    scatter_add(2 * (_PH - 1) + 1, rows1_v)

    plsc.subcore_barrier()
    @pl.when(s < _NS - 1)
    def _copy_main():
      pltpu.sync_copy(agg_sh.at[pl.ds(s * 640, 640)],
                      out_hbm.at[c, pl.ds(s * 640, 640)])

    @pl.when(s == _NS - 1)
    def _copy_tail():
      pltpu.sync_copy(agg_sh.at[pl.ds(9600, _N - 9600)],
                      out_hbm.at[c, pl.ds(9600, _N - 9600)])

  return agg_kernel


_AGGREGATE = _make_aggregate()

_BLK = 2000


def _mlp_body(p_ref, h_ref, wa_ref, ba_ref, wb_ref, bb_ref, o_ref):
  t = p_ref[0] + p_ref[1] + h_ref[...]
  t = jnp.dot(t, wa_ref[...], preferred_element_type=jnp.float32) + ba_ref[...]
  t = jnp.maximum(t, 0.0)
  t = jnp.dot(t, wb_ref[...], preferred_element_type=jnp.float32) + bb_ref[...]
  o_ref[...] = jnp.maximum(t, 0.0)


def _mlp(parts, h, wa_t, ba, wb_t, bb):
  return pl.pallas_call(
      _mlp_body,
      grid=(_N // _BLK,),
      in_specs=[
          pl.BlockSpec((_NC, _BLK, _D), lambda i: (0, i, 0)),
          pl.BlockSpec((_BLK, _D), lambda i: (i, 0)),
          pl.BlockSpec((_D, _D), lambda i: (0, 0)),
          pl.BlockSpec((1, _D), lambda i: (0, 0)),
          pl.BlockSpec((_D, _D), lambda i: (0, 0)),
          pl.BlockSpec((1, _D), lambda i: (0, 0)),
      ],
      out_specs=pl.BlockSpec((_BLK, _D), lambda i: (i, 0)),
      out_shape=jax.ShapeDtypeStruct((_N, _D), jnp.float32),
  )(parts, h, wa_t, ba, wb_t, bb)


def kernel(x, edge_index, W0, b0, W1, b1, W2, b2, W3, b3, W4, b4, W5, b5):
  eidx = jnp.stack([edge_index[0].reshape(_NTILES, _NCHUNK, _K),
                    edge_index[1].reshape(_NTILES, _NCHUNK, _K)],
                   axis=2).reshape(_NTILES, 2 * _NCHUNK, _K)
  zeros_blk = jnp.zeros((_ZROWS, _D), jnp.float32)
  h = x
  for wa, ba, wb, bb in ((W0, b0, W1, b1), (W2, b2, W3, b3), (W4, b4, W5, b5)):
    parts = _AGGREGATE(h, eidx, zeros_blk)
    h = _mlp(parts, h, wa.T, ba.reshape(1, _D), wb.T, bb.reshape(1, _D))
  return h


# speedup vs baseline: 2.2289x; 1.0095x over previous
import functools

import jax
import jax.numpy as jnp
from jax import lax
from jax.experimental import pallas as pl
from jax.experimental.pallas import tpu as pltpu
from jax.experimental.pallas import tpu_sc as plsc

_N, _E, _D = 10000, 320000, 128
_NC, _NS = 2, 16
_NTILES = _NC * _NS
_EPT = _E // _NTILES
_K = 125
_NCHUNK = _EPT // _K
_PH = _NCHUNK // 2
_ZROWS = _N // _NS


def _make_aggregate():
  mesh = plsc.VectorSubcoreMesh(core_axis_name="c", subcore_axis_name="s")

  @functools.partial(
      pl.kernel,
      mesh=mesh,
      out_type=jax.ShapeDtypeStruct((_NC, _N, _D), jnp.float32),
      scratch_types=[
          pltpu.VMEM((_PH, _K), jnp.int32),
          pltpu.VMEM((_NCHUNK, _K), jnp.int32),
          pltpu.VMEM((_K, _D), jnp.float32),
          pltpu.VMEM((_K, _D), jnp.float32),
          pltpu.VMEM_SHARED((_N, _D), jnp.float32),
          pltpu.SemaphoreType.DMA,
          pltpu.SemaphoreType.DMA,
          pltpu.SemaphoreType.DMA,
      ],
  )
  def agg_kernel(h_hbm, src_hbm, dst_hbm, zero_hbm, out_hbm,
                 src_v, dst_v, rows0_v, rows1_v, agg_sh, sem0, sem1, semz):
    c = lax.axis_index("c")
    s = lax.axis_index("s")
    wid = c * _NS + s

    def gather(jj, buf, sem):
      pltpu.async_copy(h_hbm.at[src_v.at[jj]], buf, sem)

    def drain(buf, sem):
      pltpu.make_async_copy(h_hbm.at[src_v.at[0]], buf, sem).wait()

    def scatter_add(j, buf):
      pltpu.sync_copy(buf, agg_sh.at[dst_v.at[j]], add=True)

    def pair_phase(off, npairs):
      def pair(i, carry):
        drain(rows0_v, sem0)
        scatter_add(off + 2 * i, rows0_v)
        gather(2 * i + 2, rows0_v, sem0)
        drain(rows1_v, sem1)
        scatter_add(off + 2 * i + 1, rows1_v)
        gather(2 * i + 3, rows1_v, sem1)
        return carry

      lax.fori_loop(0, npairs, pair, 0)
      drain(rows0_v, sem0)
      scatter_add(off + _PH - 2, rows0_v)
      drain(rows1_v, sem1)
      scatter_add(off + _PH - 1, rows1_v)

    src_stage = pltpu.async_copy(src_hbm.at[wid, pl.ds(0, _PH)], src_v, sem0)
    dst_stage = pltpu.async_copy(dst_hbm.at[wid], dst_v, semz)
    zero_stage = pltpu.async_copy(zero_hbm,
                                  agg_sh.at[pl.ds(s * _ZROWS, _ZROWS)], semz)
    src_stage.wait()
    gather(0, rows0_v, sem0)
    gather(1, rows1_v, sem1)
    dst_stage.wait()
    zero_stage.wait()
    plsc.subcore_barrier()

    pair_phase(0, _PH // 2 - 1)

    pltpu.sync_copy(src_hbm.at[wid, pl.ds(_PH, _PH)], src_v)
    gather(0, rows0_v, sem0)
    gather(1, rows1_v, sem1)

    pair_phase(_PH, _PH // 2 - 1)

    plsc.subcore_barrier()
    @pl.when(s < _NS - 1)
    def _copy_main():
      pltpu.sync_copy(agg_sh.at[pl.ds(s * 640, 640)],
                      out_hbm.at[c, pl.ds(s * 640, 640)])

    @pl.when(s == _NS - 1)
    def _copy_tail():
      pltpu.sync_copy(agg_sh.at[pl.ds(9600, _N - 9600)],
                      out_hbm.at[c, pl.ds(9600, _N - 9600)])

  return agg_kernel


_AGGREGATE = _make_aggregate()

_BLK = 2000


def _mlp_body(p_ref, h_ref, wa_ref, ba_ref, wb_ref, bb_ref, o_ref):
  t = p_ref[0] + p_ref[1] + h_ref[...]
  t = jnp.dot(t, wa_ref[...], preferred_element_type=jnp.float32) + ba_ref[...]
  t = jnp.maximum(t, 0.0)
  t = jnp.dot(t, wb_ref[...], preferred_element_type=jnp.float32) + bb_ref[...]
  o_ref[...] = jnp.maximum(t, 0.0)


def _mlp(parts, h, wa_t, ba, wb_t, bb):
  return pl.pallas_call(
      _mlp_body,
      grid=(_N // _BLK,),
      in_specs=[
          pl.BlockSpec((_NC, _BLK, _D), lambda i: (0, i, 0)),
          pl.BlockSpec((_BLK, _D), lambda i: (i, 0)),
          pl.BlockSpec((_D, _D), lambda i: (0, 0)),
          pl.BlockSpec((1, _D), lambda i: (0, 0)),
          pl.BlockSpec((_D, _D), lambda i: (0, 0)),
          pl.BlockSpec((1, _D), lambda i: (0, 0)),
      ],
      out_specs=pl.BlockSpec((_BLK, _D), lambda i: (i, 0)),
      out_shape=jax.ShapeDtypeStruct((_N, _D), jnp.float32),
  )(parts, h, wa_t, ba, wb_t, bb)


def kernel(x, edge_index, W0, b0, W1, b1, W2, b2, W3, b3, W4, b4, W5, b5):
  src = edge_index[0].reshape(_NTILES, _NCHUNK, _K)
  dst = edge_index[1].reshape(_NTILES, _NCHUNK, _K)
  zeros_blk = jnp.zeros((_ZROWS, _D), jnp.float32)
  h = x
  for wa, ba, wb, bb in ((W0, b0, W1, b1), (W2, b2, W3, b3), (W4, b4, W5, b5)):
    parts = _AGGREGATE(h, src, dst, zeros_blk)
    h = _mlp(parts, h, wa.T, ba.reshape(1, _D), wb.T, bb.reshape(1, _D))
  return h


# speedup vs baseline: 2.2729x; 1.0198x over previous
import functools

import jax
import jax.numpy as jnp
from jax import lax
from jax.experimental import pallas as pl
from jax.experimental.pallas import tpu as pltpu
from jax.experimental.pallas import tpu_sc as plsc

_N, _E, _D = 10000, 320000, 128
_NC, _NS = 2, 16
_NTILES = _NC * _NS
_EPT = _E // _NTILES
_K = 125
_NCHUNK = _EPT // _K
_PH = _NCHUNK // 2
_ZROWS = _N // _NS


def _make_aggregate():
  mesh = plsc.VectorSubcoreMesh(core_axis_name="c", subcore_axis_name="s")

  @functools.partial(
      pl.kernel,
      mesh=mesh,
      out_type=jax.ShapeDtypeStruct((_NC, _N, _D), jnp.float32),
      scratch_types=[
          pltpu.VMEM((_PH, _K), jnp.int32),
          pltpu.VMEM((_NCHUNK, _K), jnp.int32),
          pltpu.VMEM((_K, _D), jnp.float32),
          pltpu.VMEM((_K, _D), jnp.float32),
          pltpu.VMEM_SHARED((_N, _D), jnp.float32),
          pltpu.SemaphoreType.DMA,
          pltpu.SemaphoreType.DMA,
          pltpu.SemaphoreType.DMA,
      ],
  )
  def agg_kernel(h_hbm, src_hbm, dst_hbm, zero_hbm, out_hbm,
                 src_v, dst_v, rows0_v, rows1_v, agg_sh, sem0, sem1, semz):
    c = lax.axis_index("c")
    s = lax.axis_index("s")
    wid = c * _NS + s

    def gather(jj, buf, sem):
      pltpu.async_copy(h_hbm.at[src_v.at[jj]], buf, sem)

    def drain(buf, sem):
      pltpu.make_async_copy(h_hbm.at[src_v.at[0]], buf, sem).wait()

    def scatter_add(j, buf):
      pltpu.sync_copy(buf, agg_sh.at[dst_v.at[j]], add=True)

    def pair_phase(off, npairs):
      def pair(i, carry):
        drain(rows0_v, sem0)
        scatter_add(off + 2 * i, rows0_v)
        gather(2 * i + 2, rows0_v, sem0)
        drain(rows1_v, sem1)
        scatter_add(off + 2 * i + 1, rows1_v)
        gather(2 * i + 3, rows1_v, sem1)
        return carry

      lax.fori_loop(0, npairs, pair, 0)
      drain(rows0_v, sem0)
      scatter_add(off + _PH - 2, rows0_v)
      drain(rows1_v, sem1)
      scatter_add(off + _PH - 1, rows1_v)

    src_stage = pltpu.async_copy(src_hbm.at[wid, pl.ds(0, _PH)], src_v, sem0)
    dst_stage = pltpu.async_copy(dst_hbm.at[wid], dst_v, semz)

    @pl.when(jnp.logical_and(c == 0, s < _NS - 1))
    def _init_h_main():
      pltpu.async_copy(h_hbm.at[pl.ds(s * 640, 640)],
                       agg_sh.at[pl.ds(s * 640, 640)], semz)

    @pl.when(jnp.logical_and(c == 0, s == _NS - 1))
    def _init_h_tail():
      pltpu.async_copy(h_hbm.at[pl.ds(9600, _N - 9600)],
                       agg_sh.at[pl.ds(9600, _N - 9600)], semz)

    @pl.when(jnp.logical_and(c != 0, s < _NS - 1))
    def _init_z_main():
      pltpu.async_copy(zero_hbm, agg_sh.at[pl.ds(s * 640, 640)], semz)

    @pl.when(jnp.logical_and(c != 0, s == _NS - 1))
    def _init_z_tail():
      pltpu.async_copy(zero_hbm.at[pl.ds(0, _N - 9600)],
                       agg_sh.at[pl.ds(9600, _N - 9600)], semz)

    src_stage.wait()
    gather(0, rows0_v, sem0)
    gather(1, rows1_v, sem1)
    dst_stage.wait()

    @pl.when(s < _NS - 1)
    def _wait_init_main():
      pltpu.make_async_copy(zero_hbm, agg_sh.at[pl.ds(s * 640, 640)],
                            semz).wait()

    @pl.when(s == _NS - 1)
    def _wait_init_tail():
      pltpu.make_async_copy(zero_hbm.at[pl.ds(0, _N - 9600)],
                            agg_sh.at[pl.ds(9600, _N - 9600)], semz).wait()

    plsc.subcore_barrier()

    pair_phase(0, _PH // 2 - 1)

    pltpu.sync_copy(src_hbm.at[wid, pl.ds(_PH, _PH)], src_v)
    gather(0, rows0_v, sem0)
    gather(1, rows1_v, sem1)

    pair_phase(_PH, _PH // 2 - 1)

    plsc.subcore_barrier()
    @pl.when(s < _NS - 1)
    def _copy_main():
      pltpu.sync_copy(agg_sh.at[pl.ds(s * 640, 640)],
                      out_hbm.at[c, pl.ds(s * 640, 640)])

    @pl.when(s == _NS - 1)
    def _copy_tail():
      pltpu.sync_copy(agg_sh.at[pl.ds(9600, _N - 9600)],
                      out_hbm.at[c, pl.ds(9600, _N - 9600)])

  return agg_kernel


_AGGREGATE = _make_aggregate()

_BLK = 2000


def _mlp_body(p_ref, wa_ref, ba_ref, wb_ref, bb_ref, o_ref):
  t = p_ref[0] + p_ref[1]
  t = jnp.dot(t, wa_ref[...], preferred_element_type=jnp.float32) + ba_ref[...]
  t = jnp.maximum(t, 0.0)
  t = jnp.dot(t, wb_ref[...], preferred_element_type=jnp.float32) + bb_ref[...]
  o_ref[...] = jnp.maximum(t, 0.0)


def _mlp(parts, wa_t, ba, wb_t, bb):
  return pl.pallas_call(
      _mlp_body,
      grid=(_N // _BLK,),
      in_specs=[
          pl.BlockSpec((_NC, _BLK, _D), lambda i: (0, i, 0)),
          pl.BlockSpec((_D, _D), lambda i: (0, 0)),
          pl.BlockSpec((1, _D), lambda i: (0, 0)),
          pl.BlockSpec((_D, _D), lambda i: (0, 0)),
          pl.BlockSpec((1, _D), lambda i: (0, 0)),
      ],
      out_specs=pl.BlockSpec((_BLK, _D), lambda i: (i, 0)),
      out_shape=jax.ShapeDtypeStruct((_N, _D), jnp.float32),
  )(parts, wa_t, ba, wb_t, bb)


def kernel(x, edge_index, W0, b0, W1, b1, W2, b2, W3, b3, W4, b4, W5, b5):
  src = edge_index[0].reshape(_NTILES, _NCHUNK, _K)
  dst = edge_index[1].reshape(_NTILES, _NCHUNK, _K)
  zeros_blk = jnp.zeros((640, _D), jnp.float32)
  h = x
  for wa, ba, wb, bb in ((W0, b0, W1, b1), (W2, b2, W3, b3), (W4, b4, W5, b5)):
    parts = _AGGREGATE(h, src, dst, zeros_blk)
    h = _mlp(parts, wa.T, ba.reshape(1, _D), wb.T, bb.reshape(1, _D))
  return h
